# Initial kernel scaffold; baseline (speedup 1.0000x reference)
import functools

import jax
import jax.numpy as jnp
from jax import lax
from jax.experimental import pallas as pl
from jax.experimental.pallas import tpu as pltpu
from jax.experimental.pallas import tpu_sc as plsc

B, L, V, E, H = 1024, 50, 1000000, 64, 64
G4 = 4 * H

NC, NS = 2, 16
NW = NC * NS
NIDX = B * L
PER_W = NIDX // NW
CHUNK = 80
NCHUNK = PER_W // CHUNK


def _sc_gather(emb, idx_flat):
    mesh = plsc.VectorSubcoreMesh(core_axis_name="c", subcore_axis_name="s")

    @functools.partial(
        pl.kernel,
        mesh=mesh,
        out_type=jax.ShapeDtypeStruct((NIDX, E), jnp.float32),
        scratch_types=[
            pltpu.VMEM((CHUNK,), jnp.int32),
            pltpu.VMEM((CHUNK, E), jnp.float32),
            pltpu.SemaphoreType.DMA,
        ],
    )
    def gather_kernel(table_hbm, idx_hbm, out_hbm, idx_v, rows_v, sem):
        wid = lax.axis_index("s") * NC + lax.axis_index("c")
        base = wid * PER_W

        @pl.loop(0, NCHUNK)
        def _(j):
            off = base + j * CHUNK
            pltpu.sync_copy(idx_hbm.at[pl.ds(off, CHUNK)], idx_v)
            pltpu.async_copy(table_hbm.at[idx_v], rows_v, sem).wait()
            pltpu.sync_copy(rows_v, out_hbm.at[pl.ds(off, CHUNK)])

    return gather_kernel(emb, idx_flat)


BC = 256


def _lstm_body(e_ref, wih_ref, whh_ref, b_ref, out_ref, xp_ref):
    e2 = e_ref[...].reshape(L * BC, E)
    xp_ref[...] = (
        jnp.dot(e2, wih_ref[...], preferred_element_type=jnp.float32) + b_ref[...]
    )

    def step(t, carry):
        h, c = carry
        gates = xp_ref[pl.ds(t * BC, BC), :] + jnp.dot(
            h, whh_ref[...], preferred_element_type=jnp.float32
        )
        i = jax.nn.sigmoid(gates[:, 0:H])
        f = jax.nn.sigmoid(gates[:, H : 2 * H])
        g = jnp.tanh(gates[:, 2 * H : 3 * H])
        o = jax.nn.sigmoid(gates[:, 3 * H :])
        c = f * c + i * g
        h = o * jnp.tanh(c)
        out_ref[t] = h
        return (h, c)

    h0 = jnp.zeros((BC, H), jnp.float32)
    c0 = jnp.zeros((BC, H), jnp.float32)
    lax.fori_loop(0, L, step, (h0, c0))


def _lstm_tc(e_lbe, wih_t, whh_t, bias):
    return pl.pallas_call(
        _lstm_body,
        grid=(B // BC,),
        in_specs=[
            pl.BlockSpec((L, BC, E), lambda i: (0, i, 0)),
            pl.BlockSpec((E, G4), lambda i: (0, 0)),
            pl.BlockSpec((H, G4), lambda i: (0, 0)),
            pl.BlockSpec((1, G4), lambda i: (0, 0)),
        ],
        out_specs=pl.BlockSpec((L, BC, H), lambda i: (0, i, 0)),
        out_shape=jax.ShapeDtypeStruct((L, B, H), jnp.float32),
        scratch_shapes=[pltpu.VMEM((L * BC, G4), jnp.float32)],
    )(e_lbe, wih_t, whh_t, bias)


def kernel(x, emb, W_ih, W_hh, b_ih, b_hh):
    idx = x.T.reshape(-1).astype(jnp.int32)
    e = _sc_gather(emb, idx)
    bias = (b_ih + b_hh).reshape(1, G4)
    out_lbh = _lstm_tc(e.reshape(L, B, E), W_ih.T, W_hh.T, bias)
    return jnp.swapaxes(out_lbh, 0, 1)


# speedup vs baseline: 1.3780x; 1.3780x over previous
import functools

import jax
import jax.numpy as jnp
from jax import lax
from jax.experimental import pallas as pl
from jax.experimental.pallas import tpu as pltpu
from jax.experimental.pallas import tpu_sc as plsc

B, L, V, E, H = 1024, 50, 1000000, 64, 64
G4 = 4 * H

NC, NS = 2, 16
NW = NC * NS
NIDX = B * L
PER_W = NIDX // NW
CHUNK = 80
NCHUNK = PER_W // CHUNK


def _sc_gather(emb, idx_flat):
    mesh = plsc.VectorSubcoreMesh(core_axis_name="c", subcore_axis_name="s")

    @functools.partial(
        pl.kernel,
        mesh=mesh,
        out_type=jax.ShapeDtypeStruct((NIDX, E), jnp.float32),
        scratch_types=[
            pltpu.VMEM((CHUNK,), jnp.int32),
            pltpu.VMEM((CHUNK, E), jnp.float32),
            pltpu.SemaphoreType.DMA,
        ],
    )
    def gather_kernel(table_hbm, idx_hbm, out_hbm, idx_v, rows_v, sem):
        wid = lax.axis_index("s") * NC + lax.axis_index("c")
        base = wid * PER_W

        @pl.loop(0, NCHUNK)
        def _(j):
            off = base + j * CHUNK
            pltpu.sync_copy(idx_hbm.at[pl.ds(off, CHUNK)], idx_v)
            pltpu.async_copy(table_hbm.at[idx_v], rows_v, sem).wait()
            pltpu.sync_copy(rows_v, out_hbm.at[pl.ds(off, CHUNK)])

    return gather_kernel(emb, idx_flat)


BC = 256


def _lstm_body(e_ref, wih_ref, whh_ref, b_ref, out_ref, xp_ref):
    e2 = e_ref[...].reshape(L * BC, E)
    xp_ref[...] = (
        jnp.dot(e2, wih_ref[...], preferred_element_type=jnp.float32) + b_ref[...]
    )

    def step(t, carry):
        h, c = carry
        gates = xp_ref[pl.ds(t * BC, BC), :] + jnp.dot(
            h, whh_ref[...], preferred_element_type=jnp.float32
        )
        i = jax.nn.sigmoid(gates[:, 0:H])
        f = jax.nn.sigmoid(gates[:, H : 2 * H])
        g = jnp.tanh(gates[:, 2 * H : 3 * H])
        o = jax.nn.sigmoid(gates[:, 3 * H :])
        c = f * c + i * g
        h = o * jnp.tanh(c)
        out_ref[t] = h
        return (h, c)

    h0 = jnp.zeros((BC, H), jnp.float32)
    c0 = jnp.zeros((BC, H), jnp.float32)
    lax.fori_loop(0, L, step, (h0, c0))


def _lstm_tc(e_lbe, wih_t, whh_t, bias):
    return pl.pallas_call(
        _lstm_body,
        grid=(B // BC,),
        in_specs=[
            pl.BlockSpec((L, BC, E), lambda i: (0, i, 0)),
            pl.BlockSpec((E, G4), lambda i: (0, 0)),
            pl.BlockSpec((H, G4), lambda i: (0, 0)),
            pl.BlockSpec((1, G4), lambda i: (0, 0)),
        ],
        out_specs=pl.BlockSpec((L, BC, H), lambda i: (0, i, 0)),
        out_shape=jax.ShapeDtypeStruct((L, B, H), jnp.float32),
        scratch_shapes=[pltpu.VMEM((L * BC, G4), jnp.float32)],
    )(e_lbe, wih_t, whh_t, bias)


def kernel(x, emb, W_ih, W_hh, b_ih, b_hh):
    idx = x.T.reshape(-1).astype(jnp.int32)
    e = jnp.take(emb, idx, axis=0)
    bias = (b_ih + b_hh).reshape(1, G4)
    out_lbh = _lstm_tc(e.reshape(L, B, E), W_ih.T, W_hh.T, bias)
    return jnp.swapaxes(out_lbh, 0, 1)
